# baseline (speedup 1.0000x reference)
import functools
import jax
import jax.numpy as jnp
from jax import lax
from jax.experimental import pallas as pl
from jax.experimental.pallas import tpu as pltpu
from jax.experimental.pallas import tpu_sc as plsc

B = 16384
D = 16
NC = 2
NS = 16
L = 16
NW = NC * NS
BPW = B // NW
NCHUNK = 4
CHUNK = BPW // NCHUNK
NBLK = BPW // L


def _sc_partials(h_idxs, v_idxs, h_feats, human_table, virus_table):
    mesh = plsc.VectorSubcoreMesh(core_axis_name="c", subcore_axis_name="s")

    @functools.partial(
        pl.kernel,
        out_type=jax.ShapeDtypeStruct((NW, L), jnp.float32),
        mesh=mesh,
        compiler_params=pltpu.CompilerParams(
            needs_layout_passes=False, use_tc_tiling_on_sc=False),
        scratch_types=[
            pltpu.VMEM((NCHUNK, CHUNK), jnp.int32),
            pltpu.VMEM((NCHUNK, CHUNK), jnp.int32),
            pltpu.VMEM((BPW, D), jnp.float32),
            pltpu.VMEM((BPW, D), jnp.float32),
            pltpu.VMEM((BPW, D), jnp.float32),
            pltpu.VMEM((L, L), jnp.float32),
            pltpu.VMEM((L,), jnp.float32),
            pltpu.SemaphoreType.DMA,
            pltpu.SemaphoreType.DMA,
        ],
    )
    def sc_kernel(hidx_hbm, vidx_hbm, hf_hbm, htab_hbm, vtab_hbm, out_hbm,
                  hidx_v, vidx_v, hrows_v, vrows_v, hf_v, acc_v, t_v,
                  gsem, lsem):
        wid = lax.axis_index("s") * NC + lax.axis_index("c")

        pltpu.sync_copy(hidx_hbm.at[wid], hidx_v)
        pltpu.sync_copy(vidx_hbm.at[wid], vidx_v)
        hf_cp = pltpu.async_copy(hf_hbm.at[wid], hf_v, lsem)
        gathers = []
        for j in range(NCHUNK):
            gathers.append(pltpu.async_copy(
                htab_hbm.at[hidx_v.at[j]],
                hrows_v.at[pl.ds(j * CHUNK, CHUNK)], gsem))
            gathers.append(pltpu.async_copy(
                vtab_hbm.at[vidx_v.at[j]],
                vrows_v.at[pl.ds(j * CHUNK, CHUNK)], gsem))
        hf_cp.wait()
        for cp in gathers:
            cp.wait()

        iota = lax.iota(jnp.int32, L)
        cols = [jnp.full((L,), d, jnp.int32) for d in range(D)]
        zero = jnp.zeros((L,), jnp.float32)

        def body(kblk, accs):
            rows = kblk * L + iota
            s = zero
            for e in range(D):
                h = plsc.load_gather(hrows_v, [rows, cols[e]])
                hf = plsc.load_gather(hf_v, [rows, cols[e]])
                s = s + h * hf
            out = []
            for d in range(D):
                v = plsc.load_gather(vrows_v, [rows, cols[d]])
                out.append(accs[d] + s * v)
            return tuple(out)

        accs = lax.fori_loop(0, NBLK, body, tuple(zero for _ in range(D)))

        for d in range(D):
            acc_v[d] = accs[d]
        t = zero
        for i in range(L):
            t = t + plsc.load_gather(acc_v, [iota, cols[i]])
        t_v[...] = t
        pltpu.sync_copy(t_v, out_hbm.at[wid])

    return sc_kernel(h_idxs, v_idxs, h_feats, human_table, virus_table)


def _tc_finish(partials, v_feats_t):
    def tc_kernel(p_ref, vft_ref, o_ref):
        t = jnp.sum(p_ref[...], axis=0, keepdims=True)
        o_ref[...] = jnp.dot(t, vft_ref[...],
                             preferred_element_type=jnp.float32)

    return pl.pallas_call(
        tc_kernel,
        out_shape=jax.ShapeDtypeStruct((1, B), jnp.float32),
    )(partials, v_feats_t)


def kernel(h_idxs, v_idxs, h_feats, v_feats, human_table, virus_table):
    h_idxs = h_idxs.astype(jnp.int32).reshape(NW, NCHUNK, CHUNK)
    v_idxs = v_idxs.astype(jnp.int32).reshape(NW, NCHUNK, CHUNK)
    hf = h_feats.reshape(NW, BPW, D)
    partials = _sc_partials(h_idxs, v_idxs, hf, human_table, virus_table)
    out = _tc_finish(partials, v_feats.T)
    return out.reshape(B)
